# baseline (speedup 1.0000x reference)
import functools

import jax
import jax.numpy as jnp
from jax import lax
from jax.experimental import pallas as pl
from jax.experimental.pallas import tpu as pltpu
from jax.experimental.pallas import tpu_sc as plsc

N = 10000
E = 320000
D = 128
DH = 64

NC = 2
NS = 16
NW = NC * NS

N_PAD = 10240
E_PAD = 327680
E_ROWS = E_PAD // 128
ROWS_PER_TILE = E_PAD // NC // NS // 128
NODES_PER_TILE = N_PAD // NS

_mesh = plsc.VectorSubcoreMesh(core_axis_name="c", subcore_axis_name="s")
_sc_params = pltpu.CompilerParams(use_tc_tiling_on_sc=False)


@functools.partial(
    pl.kernel,
    out_type=jax.ShapeDtypeStruct((NC, N_PAD), jnp.float32),
    mesh=_mesh,
    scratch_types=[
        pltpu.VMEM((64,), jnp.int32),
        pltpu.VMEM((64,), jnp.float32),
        pltpu.VMEM((NODES_PER_TILE,), jnp.float32),
        pltpu.VMEM_SHARED((N_PAD,), jnp.float32),
    ],
    compiler_params=_sc_params,
)
def _sc_deg(dst2d, degp_out, idx_v, ones_v, buf_v, deg_sh):
    c = lax.axis_index("c")
    s = lax.axis_index("s")

    def fill_ones(i, _):
        ones_v[pl.ds(i * 16, 16)] = jnp.full((16,), 1.0, jnp.float32)
        return 0
    lax.fori_loop(0, 4, fill_ones, 0)

    def fill_half(i, _):
        buf_v[pl.ds(i * 16, 16)] = jnp.full((16,), 0.5, jnp.float32)
        return 0
    lax.fori_loop(0, NODES_PER_TILE // 16, fill_half, 0)
    pltpu.sync_copy(buf_v, deg_sh.at[pl.ds(s * NODES_PER_TILE, NODES_PER_TILE)])
    plsc.subcore_barrier()

    base = c * (NS * ROWS_PER_TILE * 2) + s * ROWS_PER_TILE * 2

    def body(j, _):
        pltpu.sync_copy(dst2d.at[base + j], idx_v)
        pltpu.sync_copy(ones_v, deg_sh.at[idx_v], add=True)
        return 0
    lax.fori_loop(0, ROWS_PER_TILE * 2, body, 0)
    plsc.subcore_barrier()

    pltpu.sync_copy(deg_sh.at[pl.ds(s * NODES_PER_TILE, NODES_PER_TILE)], buf_v)
    pltpu.sync_copy(buf_v, degp_out.at[c, pl.ds(s * NODES_PER_TILE, NODES_PER_TILE)])


@functools.partial(
    pl.kernel,
    out_type=jax.ShapeDtypeStruct((NC, 2, N_PAD, DH), jnp.float32),
    mesh=_mesh,
    scratch_types=[
        pltpu.VMEM((ROWS_PER_TILE * 2, 64), jnp.int32),
        pltpu.VMEM((ROWS_PER_TILE * 2, 64), jnp.int32),
        pltpu.VMEM((4, 64, DH), jnp.float32),
        pltpu.VMEM((128, DH), jnp.float32),
        pltpu.SemaphoreType.DMA,
        pltpu.SemaphoreType.DMA,
        pltpu.SemaphoreType.DMA,
        pltpu.SemaphoreType.DMA,
        pltpu.SemaphoreType.DMA,
        pltpu.SemaphoreType.DMA,
        pltpu.SemaphoreType.DMA,
        pltpu.SemaphoreType.DMA,
        pltpu.VMEM_SHARED((N_PAD, DH), jnp.float32),
    ],
    compiler_params=_sc_params,
)
def _sc_agg2(taba, tabb, src2d, dst2d, out, src_v, dst_v, bufs, zbuf,
             g0, g1, g2, g3, s0, s1, s2, s3, acc_sh):
    c = lax.axis_index("c")
    s = lax.axis_index("s")
    gs = (g0, g1, g2, g3)
    ss = (s0, s1, s2, s3)

    def zfill(i, _):
        for jj in range(DH // 16):
            zbuf[i, pl.ds(jj * 16, 16)] = jnp.zeros((16,), jnp.float32)
        return 0
    lax.fori_loop(0, 128, zfill, 0)

    NCH = ROWS_PER_TILE * 2
    nbase = s * NODES_PER_TILE
    ebase = c * (NS * NCH) + s * NCH
    pltpu.sync_copy(src2d.at[pl.ds(ebase, NCH)], src_v)
    pltpu.sync_copy(dst2d.at[pl.ds(ebase, NCH)], dst_v)

    for half, tab in ((0, taba), (1, tabb)):
        def zcopy(k, _):
            pltpu.sync_copy(zbuf, acc_sh.at[pl.ds(nbase + k * 128, 128)])
            return 0
        lax.fori_loop(0, NODES_PER_TILE // 128, zcopy, 0)
        plsc.subcore_barrier()

        def gather(j, k):
            pltpu.async_copy(tab.at[src_v.at[j]], bufs.at[k], gs[k])

        def gwait(k):
            pltpu.make_async_copy(tab.at[src_v.at[0]], bufs.at[k], gs[k]).wait()

        def scatter(j, k):
            pltpu.async_copy(bufs.at[k], acc_sh.at[dst_v.at[j]], ss[k], add=True)

        def swait(k):
            pltpu.make_async_copy(bufs.at[k], acc_sh.at[dst_v.at[0]], ss[k]).wait()

        for k in range(4):
            gather(k, k)

        def grp(g, _):
            j0 = 4 * g
            for k in range(4):
                gwait(k)
                scatter(j0 + k, k)
            for k in range(4):
                swait(k)
                gather(j0 + 4 + k, k)
            return 0
        lax.fori_loop(0, NCH // 4 - 1, grp, 0)

        j0 = NCH - 4
        for k in range(4):
            gwait(k)
            scatter(j0 + k, k)
        for k in range(4):
            swait(k)

        plsc.subcore_barrier()

        def wb(k, _):
            pltpu.sync_copy(acc_sh.at[pl.ds(nbase + k * 64, 64)], bufs.at[0])
            pltpu.sync_copy(bufs.at[0], out.at[c, half, pl.ds(nbase + k * 64, 64)])
            return 0
        lax.fori_loop(0, NODES_PER_TILE // 64, wb, 0)


_BM = 512
_GRID = N_PAD // _BM


def _tc_prep_body(degt_ref, x_ref, dis_ref, t1a_ref, t1b_ref):
    deg = degt_ref[:, 0:1] + degt_ref[:, 1:2]
    dis = lax.rsqrt(deg)
    dis_ref[...] = dis
    t1 = x_ref[...] * dis
    t1a_ref[...] = t1[:, :DH]
    t1b_ref[...] = t1[:, DH:]


def _tc_prep(degt, x_pad):
    hspec = pl.BlockSpec((_BM, DH), lambda m: (m, 0))
    return pl.pallas_call(
        _tc_prep_body,
        grid=(_GRID,),
        in_specs=[
            pl.BlockSpec((_BM, 2), lambda m: (m, 0)),
            pl.BlockSpec((_BM, D), lambda m: (m, 0)),
        ],
        out_specs=[pl.BlockSpec((_BM, 1), lambda m: (m, 0)), hspec, hspec],
        out_shape=[
            jax.ShapeDtypeStruct((N_PAD, 1), jnp.float32),
            jax.ShapeDtypeStruct((N_PAD, DH), jnp.float32),
            jax.ShapeDtypeStruct((N_PAD, DH), jnp.float32),
        ],
    )(degt, x_pad)


def _tc_layer1_body(p_ref, t1a_ref, t1b_ref, dis_ref, w1_ref, b1_ref,
                    q0_ref, q1_ref, q2_ref, q3_ref):
    dis = dis_ref[...]
    ua = (p_ref[0, 0] + p_ref[1, 0] + t1a_ref[...]) * dis
    ub = (p_ref[0, 1] + p_ref[1, 1] + t1b_ref[...]) * dis
    h = (jnp.dot(ua, w1_ref[0], preferred_element_type=jnp.float32)
         + jnp.dot(ub, w1_ref[1], preferred_element_type=jnp.float32)
         + b1_ref[...])
    h = jnp.maximum(h, 0.0)
    q0_ref[...] = h[:, :DH] * dis
    q1_ref[...] = h[:, DH:2 * DH] * dis
    q2_ref[...] = h[:, 2 * DH:3 * DH] * dis
    q3_ref[...] = h[:, 3 * DH:] * dis


def _tc_layer1(p, t1a, t1b, dis, w1, b1):
    hspec = pl.BlockSpec((_BM, DH), lambda m: (m, 0))
    hshape = jax.ShapeDtypeStruct((N_PAD, DH), jnp.float32)
    return pl.pallas_call(
        _tc_layer1_body,
        grid=(_GRID,),
        in_specs=[
            pl.BlockSpec((NC, 2, _BM, DH), lambda m: (0, 0, m, 0)),
            hspec,
            hspec,
            pl.BlockSpec((_BM, 1), lambda m: (m, 0)),
            pl.BlockSpec((2, DH, 4 * DH), lambda m: (0, 0, 0)),
            pl.BlockSpec((1, 4 * DH), lambda m: (0, 0)),
        ],
        out_specs=[hspec, hspec, hspec, hspec],
        out_shape=[hshape, hshape, hshape, hshape],
    )(p, t1a, t1b, dis, w1, b1)


def _tc_out_body(qa_ref, qb_ref, t0_ref, t1_ref, t2_ref, t3_ref, dis_ref,
                 wmu_ref, bmu_ref, wls_ref, bls_ref, mu_ref, ls_ref):
    dis = dis_ref[...]
    v0 = (qa_ref[0, 0] + qa_ref[1, 0] + t0_ref[...]) * dis
    v1 = (qa_ref[0, 1] + qa_ref[1, 1] + t1_ref[...]) * dis
    v2 = (qb_ref[0, 0] + qb_ref[1, 0] + t2_ref[...]) * dis
    v3 = (qb_ref[0, 1] + qb_ref[1, 1] + t3_ref[...]) * dis

    def mm(w_ref, b_ref):
        return (jnp.dot(v0, w_ref[0], preferred_element_type=jnp.float32)
                + jnp.dot(v1, w_ref[1], preferred_element_type=jnp.float32)
                + jnp.dot(v2, w_ref[2], preferred_element_type=jnp.float32)
                + jnp.dot(v3, w_ref[3], preferred_element_type=jnp.float32)
                + b_ref[...])

    mu_ref[...] = mm(wmu_ref, bmu_ref)
    ls_ref[...] = mm(wls_ref, bls_ref)


def _tc_out(qa, qb, t2q, dis, wmu, bmu, wls, bls):
    hspec = pl.BlockSpec((_BM, DH), lambda m: (m, 0))
    mspec = pl.BlockSpec((_BM, D), lambda m: (m, 0))
    pspec = pl.BlockSpec((NC, 2, _BM, DH), lambda m: (0, 0, m, 0))
    wspec = pl.BlockSpec((4, DH, D), lambda m: (0, 0, 0))
    bspec = pl.BlockSpec((1, D), lambda m: (0, 0))
    return pl.pallas_call(
        _tc_out_body,
        grid=(_GRID,),
        in_specs=[pspec, pspec, hspec, hspec, hspec, hspec,
                  pl.BlockSpec((_BM, 1), lambda m: (m, 0)),
                  wspec, bspec, wspec, bspec],
        out_specs=[mspec, mspec],
        out_shape=[
            jax.ShapeDtypeStruct((N_PAD, D), jnp.float32),
            jax.ShapeDtypeStruct((N_PAD, D), jnp.float32),
        ],
    )(qa, qb, *t2q, dis, wmu, bmu, wls, bls)


def kernel(x, edge_index, W1, b1, Wmu, bmu, Wls, bls):
    src = edge_index[0]
    dst = edge_index[1]
    pad = E_PAD - E
    src_pad = jnp.concatenate([src, jnp.zeros((pad,), jnp.int32)])
    dst_pad = jnp.concatenate([dst, jnp.full((pad,), N, jnp.int32)])
    src2d = src_pad.reshape(E_ROWS * 2, 64)
    dst2d = dst_pad.reshape(E_ROWS * 2, 64)
    x_pad = jnp.concatenate([x, jnp.zeros((N_PAD - N, D), x.dtype)])

    degp = _sc_deg(dst2d)
    dis, t1a, t1b = _tc_prep(degp.T, x_pad)

    p = _sc_agg2(t1a, t1b, src2d, dst2d)
    t2q = _tc_layer1(p, t1a, t1b, dis, W1.reshape(2, DH, 2 * D),
                     b1.reshape(1, 2 * D))

    qa = _sc_agg2(t2q[0], t2q[1], src2d, dst2d)
    qb = _sc_agg2(t2q[2], t2q[3], src2d, dst2d)
    mu, ls = _tc_out(qa, qb, t2q, dis,
                     Wmu.reshape(4, DH, D), bmu.reshape(1, D),
                     Wls.reshape(4, DH, D), bls.reshape(1, D))
    return (mu[:N], ls[:N])


# speedup vs baseline: 1.5388x; 1.5388x over previous
import functools

import jax
import jax.numpy as jnp
from jax import lax
from jax.experimental import pallas as pl
from jax.experimental.pallas import tpu as pltpu
from jax.experimental.pallas import tpu_sc as plsc

N = 10000
E = 320000
D = 128
DH = 64

NC = 2
NS = 16
NW = NC * NS

N_PAD = 10240
E_PAD = 327680
E_ROWS = E_PAD // 128
ROWS_PER_TILE = E_PAD // NC // NS // 128
NODES_PER_TILE = N_PAD // NS

_mesh = plsc.VectorSubcoreMesh(core_axis_name="c", subcore_axis_name="s")
_sc_params = pltpu.CompilerParams(use_tc_tiling_on_sc=False)


@functools.partial(
    pl.kernel,
    out_type=jax.ShapeDtypeStruct((NC, N_PAD), jnp.float32),
    mesh=_mesh,
    scratch_types=[
        pltpu.VMEM((64,), jnp.int32),
        pltpu.VMEM((64,), jnp.float32),
        pltpu.VMEM((NODES_PER_TILE,), jnp.float32),
        pltpu.VMEM_SHARED((N_PAD,), jnp.float32),
    ],
    compiler_params=_sc_params,
)
def _sc_deg(dst2d, degp_out, idx_v, ones_v, buf_v, deg_sh):
    c = lax.axis_index("c")
    s = lax.axis_index("s")

    def fill_ones(i, _):
        ones_v[pl.ds(i * 16, 16)] = jnp.full((16,), 1.0, jnp.float32)
        return 0
    lax.fori_loop(0, 4, fill_ones, 0)

    def fill_half(i, _):
        buf_v[pl.ds(i * 16, 16)] = jnp.full((16,), 0.5, jnp.float32)
        return 0
    lax.fori_loop(0, NODES_PER_TILE // 16, fill_half, 0)
    pltpu.sync_copy(buf_v, deg_sh.at[pl.ds(s * NODES_PER_TILE, NODES_PER_TILE)])
    plsc.subcore_barrier()

    base = c * (NS * ROWS_PER_TILE * 2) + s * ROWS_PER_TILE * 2

    def body(j, _):
        pltpu.sync_copy(dst2d.at[base + j], idx_v)
        pltpu.sync_copy(ones_v, deg_sh.at[idx_v], add=True)
        return 0
    lax.fori_loop(0, ROWS_PER_TILE * 2, body, 0)
    plsc.subcore_barrier()

    pltpu.sync_copy(deg_sh.at[pl.ds(s * NODES_PER_TILE, NODES_PER_TILE)], buf_v)
    pltpu.sync_copy(buf_v, degp_out.at[c, pl.ds(s * NODES_PER_TILE, NODES_PER_TILE)])


@functools.partial(
    pl.kernel,
    out_type=jax.ShapeDtypeStruct((NC, 2, N_PAD, DH), jnp.bfloat16),
    mesh=_mesh,
    scratch_types=[
        pltpu.VMEM((ROWS_PER_TILE * 2, 64), jnp.int32),
        pltpu.VMEM((ROWS_PER_TILE * 2, 64), jnp.int32),
        pltpu.VMEM((4, 64, DH), jnp.bfloat16),
        pltpu.VMEM((128, DH), jnp.bfloat16),
        pltpu.SemaphoreType.DMA,
        pltpu.SemaphoreType.DMA,
        pltpu.SemaphoreType.DMA,
        pltpu.SemaphoreType.DMA,
        pltpu.SemaphoreType.DMA,
        pltpu.SemaphoreType.DMA,
        pltpu.SemaphoreType.DMA,
        pltpu.SemaphoreType.DMA,
        pltpu.VMEM_SHARED((N_PAD, DH), jnp.bfloat16),
    ],
    compiler_params=_sc_params,
)
def _sc_agg2(taba, tabb, src2d, dst2d, out, src_v, dst_v, bufs, zbuf,
             g0, g1, g2, g3, s0, s1, s2, s3, acc_sh):
    c = lax.axis_index("c")
    s = lax.axis_index("s")
    gs = (g0, g1, g2, g3)
    ss = (s0, s1, s2, s3)

    def zfill(i, _):
        for jj in range(DH // 32):
            zbuf[i, pl.ds(jj * 32, 32)] = jnp.zeros((32,), jnp.bfloat16)
        return 0
    lax.fori_loop(0, 128, zfill, 0)

    NCH = ROWS_PER_TILE * 2
    nbase = s * NODES_PER_TILE
    ebase = c * (NS * NCH) + s * NCH
    pltpu.sync_copy(src2d.at[pl.ds(ebase, NCH)], src_v)
    pltpu.sync_copy(dst2d.at[pl.ds(ebase, NCH)], dst_v)

    for half, tab in ((0, taba), (1, tabb)):
        def zcopy(k, _):
            pltpu.sync_copy(zbuf, acc_sh.at[pl.ds(nbase + k * 128, 128)])
            return 0
        lax.fori_loop(0, NODES_PER_TILE // 128, zcopy, 0)
        plsc.subcore_barrier()

        def gather(j, k):
            pltpu.async_copy(tab.at[src_v.at[j]], bufs.at[k], gs[k])

        def gwait(k):
            pltpu.make_async_copy(tab.at[src_v.at[0]], bufs.at[k], gs[k]).wait()

        def scatter(j, k):
            pltpu.async_copy(bufs.at[k], acc_sh.at[dst_v.at[j]], ss[k], add=True)

        def swait(k):
            pltpu.make_async_copy(bufs.at[k], acc_sh.at[dst_v.at[0]], ss[k]).wait()

        for k in range(4):
            gather(k, k)

        def grp(g, _):
            j0 = 4 * g
            for k in range(4):
                gwait(k)
                scatter(j0 + k, k)
            for k in range(4):
                swait(k)
                gather(j0 + 4 + k, k)
            return 0
        lax.fori_loop(0, NCH // 4 - 1, grp, 0)

        j0 = NCH - 4
        for k in range(4):
            gwait(k)
            scatter(j0 + k, k)
        for k in range(4):
            swait(k)

        plsc.subcore_barrier()

        def wb(k, _):
            pltpu.sync_copy(acc_sh.at[pl.ds(nbase + k * 64, 64)], bufs.at[0])
            pltpu.sync_copy(bufs.at[0], out.at[c, half, pl.ds(nbase + k * 64, 64)])
            return 0
        lax.fori_loop(0, NODES_PER_TILE // 64, wb, 0)


_BM = 512
_GRID = N_PAD // _BM


def _tc_prep_body(degt_ref, x_ref, dis_ref, t1a_ref, t1b_ref):
    deg = degt_ref[:, 0:1] + degt_ref[:, 1:2]
    dis = lax.rsqrt(deg)
    dis_ref[...] = dis
    t1 = x_ref[...] * dis
    t1a_ref[...] = t1[:, :DH].astype(jnp.bfloat16)
    t1b_ref[...] = t1[:, DH:].astype(jnp.bfloat16)


def _tc_prep(degt, x_pad):
    hspec = pl.BlockSpec((_BM, DH), lambda m: (m, 0))
    return pl.pallas_call(
        _tc_prep_body,
        grid=(_GRID,),
        in_specs=[
            pl.BlockSpec((_BM, 2), lambda m: (m, 0)),
            pl.BlockSpec((_BM, D), lambda m: (m, 0)),
        ],
        out_specs=[pl.BlockSpec((_BM, 1), lambda m: (m, 0)), hspec, hspec],
        out_shape=[
            jax.ShapeDtypeStruct((N_PAD, 1), jnp.float32),
            jax.ShapeDtypeStruct((N_PAD, DH), jnp.bfloat16),
            jax.ShapeDtypeStruct((N_PAD, DH), jnp.bfloat16),
        ],
    )(degt, x_pad)


def _tc_layer1_body(p_ref, t1a_ref, t1b_ref, dis_ref, w1_ref, b1_ref,
                    q0_ref, q1_ref, q2_ref, q3_ref):
    dis = dis_ref[...]
    f = lambda x: x.astype(jnp.float32)
    ua = (f(p_ref[0, 0]) + f(p_ref[1, 0]) + f(t1a_ref[...])) * dis
    ub = (f(p_ref[0, 1]) + f(p_ref[1, 1]) + f(t1b_ref[...])) * dis
    h = (jnp.dot(ua, w1_ref[0], preferred_element_type=jnp.float32)
         + jnp.dot(ub, w1_ref[1], preferred_element_type=jnp.float32)
         + b1_ref[...])
    h = jnp.maximum(h, 0.0)
    q0_ref[...] = (h[:, :DH] * dis).astype(jnp.bfloat16)
    q1_ref[...] = (h[:, DH:2 * DH] * dis).astype(jnp.bfloat16)
    q2_ref[...] = (h[:, 2 * DH:3 * DH] * dis).astype(jnp.bfloat16)
    q3_ref[...] = (h[:, 3 * DH:] * dis).astype(jnp.bfloat16)


def _tc_layer1(p, t1a, t1b, dis, w1, b1):
    hspec = pl.BlockSpec((_BM, DH), lambda m: (m, 0))
    hshape = jax.ShapeDtypeStruct((N_PAD, DH), jnp.bfloat16)
    return pl.pallas_call(
        _tc_layer1_body,
        grid=(_GRID,),
        in_specs=[
            pl.BlockSpec((NC, 2, _BM, DH), lambda m: (0, 0, m, 0)),
            hspec,
            hspec,
            pl.BlockSpec((_BM, 1), lambda m: (m, 0)),
            pl.BlockSpec((2, DH, 4 * DH), lambda m: (0, 0, 0)),
            pl.BlockSpec((1, 4 * DH), lambda m: (0, 0)),
        ],
        out_specs=[hspec, hspec, hspec, hspec],
        out_shape=[hshape, hshape, hshape, hshape],
    )(p, t1a, t1b, dis, w1, b1)


def _tc_out_body(qa_ref, qb_ref, t0_ref, t1_ref, t2_ref, t3_ref, dis_ref,
                 wmu_ref, bmu_ref, wls_ref, bls_ref, mu_ref, ls_ref):
    dis = dis_ref[...]
    f = lambda x: x.astype(jnp.float32)
    v0 = (f(qa_ref[0, 0]) + f(qa_ref[1, 0]) + f(t0_ref[...])) * dis
    v1 = (f(qa_ref[0, 1]) + f(qa_ref[1, 1]) + f(t1_ref[...])) * dis
    v2 = (f(qb_ref[0, 0]) + f(qb_ref[1, 0]) + f(t2_ref[...])) * dis
    v3 = (f(qb_ref[0, 1]) + f(qb_ref[1, 1]) + f(t3_ref[...])) * dis

    def mm(w_ref, b_ref):
        return (jnp.dot(v0, w_ref[0], preferred_element_type=jnp.float32)
                + jnp.dot(v1, w_ref[1], preferred_element_type=jnp.float32)
                + jnp.dot(v2, w_ref[2], preferred_element_type=jnp.float32)
                + jnp.dot(v3, w_ref[3], preferred_element_type=jnp.float32)
                + b_ref[...])

    mu_ref[...] = mm(wmu_ref, bmu_ref)
    ls_ref[...] = mm(wls_ref, bls_ref)


def _tc_out(qa, qb, t2q, dis, wmu, bmu, wls, bls):
    hspec = pl.BlockSpec((_BM, DH), lambda m: (m, 0))
    mspec = pl.BlockSpec((_BM, D), lambda m: (m, 0))
    pspec = pl.BlockSpec((NC, 2, _BM, DH), lambda m: (0, 0, m, 0))
    wspec = pl.BlockSpec((4, DH, D), lambda m: (0, 0, 0))
    bspec = pl.BlockSpec((1, D), lambda m: (0, 0))
    return pl.pallas_call(
        _tc_out_body,
        grid=(_GRID,),
        in_specs=[pspec, pspec, hspec, hspec, hspec, hspec,
                  pl.BlockSpec((_BM, 1), lambda m: (m, 0)),
                  wspec, bspec, wspec, bspec],
        out_specs=[mspec, mspec],
        out_shape=[
            jax.ShapeDtypeStruct((N_PAD, D), jnp.float32),
            jax.ShapeDtypeStruct((N_PAD, D), jnp.float32),
        ],
    )(qa, qb, *t2q, dis, wmu, bmu, wls, bls)


def kernel(x, edge_index, W1, b1, Wmu, bmu, Wls, bls):
    src = edge_index[0]
    dst = edge_index[1]
    pad = E_PAD - E
    src_pad = jnp.concatenate([src, jnp.zeros((pad,), jnp.int32)])
    dst_pad = jnp.concatenate([dst, jnp.full((pad,), N, jnp.int32)])
    src2d = src_pad.reshape(E_ROWS * 2, 64)
    dst2d = dst_pad.reshape(E_ROWS * 2, 64)
    x_pad = jnp.concatenate([x, jnp.zeros((N_PAD - N, D), x.dtype)])

    degp = _sc_deg(dst2d)
    dis, t1a, t1b = _tc_prep(degp.T, x_pad)

    p = _sc_agg2(t1a, t1b, src2d, dst2d)
    t2q = _tc_layer1(p, t1a, t1b, dis, W1.reshape(2, DH, 2 * D),
                     b1.reshape(1, 2 * D))

    qa = _sc_agg2(t2q[0], t2q[1], src2d, dst2d)
    qb = _sc_agg2(t2q[2], t2q[3], src2d, dst2d)
    mu, ls = _tc_out(qa, qb, t2q, dis,
                     Wmu.reshape(4, DH, D), bmu.reshape(1, D),
                     Wls.reshape(4, DH, D), bls.reshape(1, D))
    return (mu[:N], ls[:N])


# speedup vs baseline: 1.6861x; 1.0957x over previous
import functools

import jax
import jax.numpy as jnp
from jax import lax
from jax.experimental import pallas as pl
from jax.experimental.pallas import tpu as pltpu
from jax.experimental.pallas import tpu_sc as plsc

N = 10000
E = 320000
D = 128

NC = 2
NS = 16
NW = NC * NS

N_PAD = 10240
E_PAD = 327680
E_CHUNKS = E_PAD // 64
CH_PER_TILE = E_PAD // NC // NS // 64
NODES_PER_TILE = N_PAD // NS

_mesh = plsc.VectorSubcoreMesh(core_axis_name="c", subcore_axis_name="s")
_sc_params = pltpu.CompilerParams(use_tc_tiling_on_sc=False)


@functools.partial(
    pl.kernel,
    out_type=jax.ShapeDtypeStruct((NC, N_PAD), jnp.float32),
    mesh=_mesh,
    scratch_types=[
        pltpu.VMEM((64,), jnp.int32),
        pltpu.VMEM((64,), jnp.float32),
        pltpu.VMEM((NODES_PER_TILE,), jnp.float32),
        pltpu.VMEM_SHARED((N_PAD,), jnp.float32),
    ],
    compiler_params=_sc_params,
)
def _sc_deg(dst2d, degp_out, idx_v, ones_v, buf_v, deg_sh):
    c = lax.axis_index("c")
    s = lax.axis_index("s")

    def fill_ones(i, _):
        ones_v[pl.ds(i * 16, 16)] = jnp.full((16,), 1.0, jnp.float32)
        return 0
    lax.fori_loop(0, 4, fill_ones, 0)

    def fill_half(i, _):
        buf_v[pl.ds(i * 16, 16)] = jnp.full((16,), 0.5, jnp.float32)
        return 0
    lax.fori_loop(0, NODES_PER_TILE // 16, fill_half, 0)
    pltpu.sync_copy(buf_v, deg_sh.at[pl.ds(s * NODES_PER_TILE, NODES_PER_TILE)])
    plsc.subcore_barrier()

    base = c * (NS * CH_PER_TILE) + s * CH_PER_TILE

    def body(j, _):
        pltpu.sync_copy(dst2d.at[base + j], idx_v)
        pltpu.sync_copy(ones_v, deg_sh.at[idx_v], add=True)
        return 0
    lax.fori_loop(0, CH_PER_TILE, body, 0)
    plsc.subcore_barrier()

    pltpu.sync_copy(deg_sh.at[pl.ds(s * NODES_PER_TILE, NODES_PER_TILE)], buf_v)
    pltpu.sync_copy(buf_v, degp_out.at[c, pl.ds(s * NODES_PER_TILE, NODES_PER_TILE)])


def _agg_pass(tab, out_slot, c, s, src_v, dst_v, bufs, zbuf, gs, ss, acc_sh):
    nbase = s * NODES_PER_TILE

    def zcopy(k, _):
        pltpu.sync_copy(zbuf, acc_sh.at[pl.ds(nbase + k * 128, 128)])
        return 0
    lax.fori_loop(0, NODES_PER_TILE // 128, zcopy, 0)
    plsc.subcore_barrier()

    def gather(j, k):
        pltpu.async_copy(tab.at[src_v.at[j]], bufs.at[k], gs[k])

    def gwait(k):
        pltpu.make_async_copy(tab.at[src_v.at[0]], bufs.at[k], gs[k]).wait()

    def scatter(j, k):
        pltpu.async_copy(bufs.at[k], acc_sh.at[dst_v.at[j]], ss[k], add=True)

    def swait(k):
        pltpu.make_async_copy(bufs.at[k], acc_sh.at[dst_v.at[0]], ss[k]).wait()

    for k in range(4):
        gather(k, k)

    def grp(g, _):
        j0 = 4 * g
        for k in range(4):
            gwait(k)
            scatter(j0 + k, k)
        for k in range(4):
            swait(k)
            gather(j0 + 4 + k, k)
        return 0
    lax.fori_loop(0, CH_PER_TILE // 4 - 1, grp, 0)

    j0 = CH_PER_TILE - 4
    for k in range(4):
        gwait(k)
        scatter(j0 + k, k)
    for k in range(4):
        swait(k)

    plsc.subcore_barrier()

    def wb(k, _):
        pltpu.sync_copy(acc_sh.at[pl.ds(nbase + k * 64, 64)], bufs.at[0])
        pltpu.sync_copy(bufs.at[0], out_slot.at[pl.ds(nbase + k * 64, 64)])
        return 0
    lax.fori_loop(0, NODES_PER_TILE // 64, wb, 0)


_AGG_SCRATCH = [
    pltpu.VMEM((CH_PER_TILE, 64), jnp.int32),
    pltpu.VMEM((CH_PER_TILE, 64), jnp.int32),
    pltpu.VMEM((4, 64, D), jnp.bfloat16),
    pltpu.VMEM((128, D), jnp.bfloat16),
    pltpu.SemaphoreType.DMA,
    pltpu.SemaphoreType.DMA,
    pltpu.SemaphoreType.DMA,
    pltpu.SemaphoreType.DMA,
    pltpu.SemaphoreType.DMA,
    pltpu.SemaphoreType.DMA,
    pltpu.SemaphoreType.DMA,
    pltpu.SemaphoreType.DMA,
    pltpu.VMEM_SHARED((N_PAD, D), jnp.bfloat16),
]


def _agg_prologue(c, s, src2d, dst2d, src_v, dst_v, zbuf):
    def zfill(i, _):
        for jj in range(D // 32):
            zbuf[i, pl.ds(jj * 32, 32)] = jnp.zeros((32,), jnp.bfloat16)
        return 0
    lax.fori_loop(0, 128, zfill, 0)

    ebase = c * (NS * CH_PER_TILE) + s * CH_PER_TILE
    pltpu.sync_copy(src2d.at[pl.ds(ebase, CH_PER_TILE)], src_v)
    pltpu.sync_copy(dst2d.at[pl.ds(ebase, CH_PER_TILE)], dst_v)


@functools.partial(
    pl.kernel,
    out_type=jax.ShapeDtypeStruct((NC, N_PAD, D), jnp.bfloat16),
    mesh=_mesh,
    scratch_types=_AGG_SCRATCH,
    compiler_params=_sc_params,
)
def _sc_agg1(tab, src2d, dst2d, out, src_v, dst_v, bufs, zbuf,
             g0, g1, g2, g3, s0, s1, s2, s3, acc_sh):
    c = lax.axis_index("c")
    s = lax.axis_index("s")
    _agg_prologue(c, s, src2d, dst2d, src_v, dst_v, zbuf)
    _agg_pass(tab, out.at[c], c, s, src_v, dst_v, bufs, zbuf,
              (g0, g1, g2, g3), (s0, s1, s2, s3), acc_sh)


@functools.partial(
    pl.kernel,
    out_type=jax.ShapeDtypeStruct((NC, 2, N_PAD, D), jnp.bfloat16),
    mesh=_mesh,
    scratch_types=_AGG_SCRATCH,
    compiler_params=_sc_params,
)
def _sc_agg2(taba, tabb, src2d, dst2d, out, src_v, dst_v, bufs, zbuf,
             g0, g1, g2, g3, s0, s1, s2, s3, acc_sh):
    c = lax.axis_index("c")
    s = lax.axis_index("s")
    _agg_prologue(c, s, src2d, dst2d, src_v, dst_v, zbuf)
    for half, tab in ((0, taba), (1, tabb)):
        _agg_pass(tab, out.at[c, half], c, s, src_v, dst_v, bufs, zbuf,
                  (g0, g1, g2, g3), (s0, s1, s2, s3), acc_sh)


_BM = 512
_GRID = N_PAD // _BM


def _f32(x):
    return x.astype(jnp.float32)


def _tc_prep_body(degt_ref, x_ref, dis_ref, t1_ref):
    deg = degt_ref[:, 0:1] + degt_ref[:, 1:2]
    dis = lax.rsqrt(deg)
    dis_ref[...] = dis
    t1_ref[...] = (x_ref[...] * dis).astype(jnp.bfloat16)


def _tc_prep(degt, x_pad):
    return pl.pallas_call(
        _tc_prep_body,
        grid=(_GRID,),
        in_specs=[
            pl.BlockSpec((_BM, 2), lambda m: (m, 0)),
            pl.BlockSpec((_BM, D), lambda m: (m, 0)),
        ],
        out_specs=[
            pl.BlockSpec((_BM, 1), lambda m: (m, 0)),
            pl.BlockSpec((_BM, D), lambda m: (m, 0)),
        ],
        out_shape=[
            jax.ShapeDtypeStruct((N_PAD, 1), jnp.float32),
            jax.ShapeDtypeStruct((N_PAD, D), jnp.bfloat16),
        ],
    )(degt, x_pad)


def _tc_layer1_body(p_ref, t1_ref, dis_ref, w1_ref, b1_ref, t2a_ref, t2b_ref):
    dis = dis_ref[...]
    u = (_f32(p_ref[0]) + _f32(p_ref[1]) + _f32(t1_ref[...])) * dis
    h = jnp.dot(u, w1_ref[...], preferred_element_type=jnp.float32) + b1_ref[...]
    h = jnp.maximum(h, 0.0)
    t2a_ref[...] = (h[:, :D] * dis).astype(jnp.bfloat16)
    t2b_ref[...] = (h[:, D:] * dis).astype(jnp.bfloat16)


def _tc_layer1(p, t1, dis, w1, b1):
    hspec = pl.BlockSpec((_BM, D), lambda m: (m, 0))
    hshape = jax.ShapeDtypeStruct((N_PAD, D), jnp.bfloat16)
    return pl.pallas_call(
        _tc_layer1_body,
        grid=(_GRID,),
        in_specs=[
            pl.BlockSpec((NC, _BM, D), lambda m: (0, m, 0)),
            hspec,
            pl.BlockSpec((_BM, 1), lambda m: (m, 0)),
            pl.BlockSpec((D, 2 * D), lambda m: (0, 0)),
            pl.BlockSpec((1, 2 * D), lambda m: (0, 0)),
        ],
        out_specs=[hspec, hspec],
        out_shape=[hshape, hshape],
    )(p, t1, dis, w1, b1)


def _tc_out_body(q_ref, t2a_ref, t2b_ref, dis_ref,
                 wmu_ref, bmu_ref, wls_ref, bls_ref, mu_ref, ls_ref):
    dis = dis_ref[...]
    va = (_f32(q_ref[0, 0]) + _f32(q_ref[1, 0]) + _f32(t2a_ref[...])) * dis
    vb = (_f32(q_ref[0, 1]) + _f32(q_ref[1, 1]) + _f32(t2b_ref[...])) * dis

    def mm(w_ref, b_ref):
        return (jnp.dot(va, w_ref[0], preferred_element_type=jnp.float32)
                + jnp.dot(vb, w_ref[1], preferred_element_type=jnp.float32)
                + b_ref[...])

    mu_ref[...] = mm(wmu_ref, bmu_ref)
    ls_ref[...] = mm(wls_ref, bls_ref)


def _tc_out(q, t2a, t2b, dis, wmu, bmu, wls, bls):
    hspec = pl.BlockSpec((_BM, D), lambda m: (m, 0))
    wspec = pl.BlockSpec((2, D, D), lambda m: (0, 0, 0))
    bspec = pl.BlockSpec((1, D), lambda m: (0, 0))
    return pl.pallas_call(
        _tc_out_body,
        grid=(_GRID,),
        in_specs=[
            pl.BlockSpec((NC, 2, _BM, D), lambda m: (0, 0, m, 0)),
            hspec,
            hspec,
            pl.BlockSpec((_BM, 1), lambda m: (m, 0)),
            wspec, bspec, wspec, bspec,
        ],
        out_specs=[hspec, hspec],
        out_shape=[
            jax.ShapeDtypeStruct((N_PAD, D), jnp.float32),
            jax.ShapeDtypeStruct((N_PAD, D), jnp.float32),
        ],
    )(q, t2a, t2b, dis, wmu, bmu, wls, bls)


def kernel(x, edge_index, W1, b1, Wmu, bmu, Wls, bls):
    src = edge_index[0]
    dst = edge_index[1]
    pad = E_PAD - E
    src_pad = jnp.concatenate([src, jnp.zeros((pad,), jnp.int32)])
    dst_pad = jnp.concatenate([dst, jnp.full((pad,), N, jnp.int32)])
    src2d = src_pad.reshape(E_CHUNKS, 64)
    dst2d = dst_pad.reshape(E_CHUNKS, 64)
    x_pad = jnp.concatenate([x, jnp.zeros((N_PAD - N, D), x.dtype)])

    degp = _sc_deg(dst2d)
    dis, t1 = _tc_prep(degp.T, x_pad)

    p = _sc_agg1(t1, src2d, dst2d)
    t2a, t2b = _tc_layer1(p, t1, dis, W1, b1.reshape(1, 2 * D))

    q = _sc_agg2(t2a, t2b, src2d, dst2d)
    mu, ls = _tc_out(q, t2a, t2b, dis,
                     Wmu.reshape(2, D, D), bmu.reshape(1, D),
                     Wls.reshape(2, D, D), bls.reshape(1, D))
    return (mu[:N], ls[:N])


# speedup vs baseline: 1.7929x; 1.0634x over previous
import functools

import jax
import jax.numpy as jnp
from jax import lax
from jax.experimental import pallas as pl
from jax.experimental.pallas import tpu as pltpu
from jax.experimental.pallas import tpu_sc as plsc

N = 10000
E = 320000
D = 128

NC = 2
NS = 16
NW = NC * NS

N_PAD = 10240
E_PAD = 327680
E_CHUNKS = E_PAD // 64
CH_PER_TILE = E_PAD // NC // NS // 64
NODES_PER_TILE = N_PAD // NS

_mesh = plsc.VectorSubcoreMesh(core_axis_name="c", subcore_axis_name="s")
_sc_params = pltpu.CompilerParams(use_tc_tiling_on_sc=False)


@functools.partial(
    pl.kernel,
    out_type=jax.ShapeDtypeStruct((NC, N_PAD), jnp.float32),
    mesh=_mesh,
    scratch_types=[
        pltpu.VMEM((4, 64), jnp.int32),
        pltpu.VMEM((64,), jnp.float32),
        pltpu.VMEM((NODES_PER_TILE,), jnp.float32),
        pltpu.SemaphoreType.DMA,
        pltpu.SemaphoreType.DMA,
        pltpu.SemaphoreType.DMA,
        pltpu.SemaphoreType.DMA,
        pltpu.SemaphoreType.DMA,
        pltpu.SemaphoreType.DMA,
        pltpu.SemaphoreType.DMA,
        pltpu.SemaphoreType.DMA,
        pltpu.VMEM_SHARED((N_PAD,), jnp.float32),
    ],
    compiler_params=_sc_params,
)
def _sc_deg(dst2d, degp_out, idx_v, ones_v, buf_v,
            g0, g1, g2, g3, s0, s1, s2, s3, deg_sh):
    c = lax.axis_index("c")
    s = lax.axis_index("s")
    gs = (g0, g1, g2, g3)
    ss = (s0, s1, s2, s3)

    def fill_ones(i, _):
        ones_v[pl.ds(i * 16, 16)] = jnp.full((16,), 1.0, jnp.float32)
        return 0
    lax.fori_loop(0, 4, fill_ones, 0)

    def fill_half(i, _):
        buf_v[pl.ds(i * 16, 16)] = jnp.full((16,), 0.5, jnp.float32)
        return 0
    lax.fori_loop(0, NODES_PER_TILE // 16, fill_half, 0)
    pltpu.sync_copy(buf_v, deg_sh.at[pl.ds(s * NODES_PER_TILE, NODES_PER_TILE)])
    plsc.subcore_barrier()

    base = c * (NS * CH_PER_TILE) + s * CH_PER_TILE

    def iload(j, k):
        pltpu.async_copy(dst2d.at[base + j], idx_v.at[k], gs[k])

    def iwait(k):
        pltpu.make_async_copy(dst2d.at[base], idx_v.at[k], gs[k]).wait()

    def scat(k):
        pltpu.async_copy(ones_v, deg_sh.at[idx_v.at[k]], ss[k], add=True)

    def swaitd(k):
        pltpu.make_async_copy(ones_v, deg_sh.at[idx_v.at[k]], ss[k]).wait()

    for k in range(4):
        iload(k, k)

    def grp(g, _):
        j0 = 4 * g
        for k in range(4):
            iwait(k)
            scat(k)
        for k in range(4):
            swaitd(k)
            iload(j0 + 4 + k, k)
        return 0
    lax.fori_loop(0, CH_PER_TILE // 4 - 1, grp, 0)

    for k in range(4):
        iwait(k)
        scat(k)
    for k in range(4):
        swaitd(k)
    plsc.subcore_barrier()

    pltpu.sync_copy(deg_sh.at[pl.ds(s * NODES_PER_TILE, NODES_PER_TILE)], buf_v)
    pltpu.sync_copy(buf_v, degp_out.at[c, pl.ds(s * NODES_PER_TILE, NODES_PER_TILE)])


def _agg_pass(tab, out_slot, c, s, src_v, dst_v, bufs, zbuf, gs, ss, acc_sh):
    nbase = s * NODES_PER_TILE

    def zcopy(k, _):
        pltpu.sync_copy(zbuf, acc_sh.at[pl.ds(nbase + k * 128, 128)])
        return 0
    lax.fori_loop(0, NODES_PER_TILE // 128, zcopy, 0)
    plsc.subcore_barrier()

    def gather(j, k):
        pltpu.async_copy(tab.at[src_v.at[j]], bufs.at[k], gs[k])

    def gwait(k):
        pltpu.make_async_copy(tab.at[src_v.at[0]], bufs.at[k], gs[k]).wait()

    def scatter(j, k):
        pltpu.async_copy(bufs.at[k], acc_sh.at[dst_v.at[j]], ss[k], add=True)

    def swait(k):
        pltpu.make_async_copy(bufs.at[k], acc_sh.at[dst_v.at[0]], ss[k]).wait()

    for k in range(4):
        gather(k, k)

    def grp(g, _):
        j0 = 4 * g
        for k in range(4):
            gwait(k)
            scatter(j0 + k, k)
        for k in range(4):
            swait(k)
            gather(j0 + 4 + k, k)
        return 0
    lax.fori_loop(0, CH_PER_TILE // 4 - 1, grp, 0)

    j0 = CH_PER_TILE - 4
    for k in range(4):
        gwait(k)
        scatter(j0 + k, k)
    for k in range(4):
        swait(k)

    plsc.subcore_barrier()

    def wb(k, _):
        pltpu.sync_copy(acc_sh.at[pl.ds(nbase + k * 64, 64)], bufs.at[0])
        pltpu.sync_copy(bufs.at[0], out_slot.at[pl.ds(nbase + k * 64, 64)])
        return 0
    lax.fori_loop(0, NODES_PER_TILE // 64, wb, 0)


_AGG_SCRATCH = [
    pltpu.VMEM((CH_PER_TILE, 64), jnp.int32),
    pltpu.VMEM((CH_PER_TILE, 64), jnp.int32),
    pltpu.VMEM((4, 64, D), jnp.bfloat16),
    pltpu.VMEM((128, D), jnp.bfloat16),
    pltpu.SemaphoreType.DMA,
    pltpu.SemaphoreType.DMA,
    pltpu.SemaphoreType.DMA,
    pltpu.SemaphoreType.DMA,
    pltpu.SemaphoreType.DMA,
    pltpu.SemaphoreType.DMA,
    pltpu.SemaphoreType.DMA,
    pltpu.SemaphoreType.DMA,
    pltpu.VMEM_SHARED((N_PAD, D), jnp.bfloat16),
]


def _agg_prologue(c, s, src2d, dst2d, src_v, dst_v, zbuf):
    def zfill(i, _):
        for jj in range(D // 32):
            zbuf[i, pl.ds(jj * 32, 32)] = jnp.zeros((32,), jnp.bfloat16)
        return 0
    lax.fori_loop(0, 128, zfill, 0)

    ebase = c * (NS * CH_PER_TILE) + s * CH_PER_TILE
    pltpu.sync_copy(src2d.at[pl.ds(ebase, CH_PER_TILE)], src_v)
    pltpu.sync_copy(dst2d.at[pl.ds(ebase, CH_PER_TILE)], dst_v)


@functools.partial(
    pl.kernel,
    out_type=jax.ShapeDtypeStruct((NC, N_PAD, D), jnp.bfloat16),
    mesh=_mesh,
    scratch_types=_AGG_SCRATCH,
    compiler_params=_sc_params,
)
def _sc_agg1(tab, src2d, dst2d, out, src_v, dst_v, bufs, zbuf,
             g0, g1, g2, g3, s0, s1, s2, s3, acc_sh):
    c = lax.axis_index("c")
    s = lax.axis_index("s")
    _agg_prologue(c, s, src2d, dst2d, src_v, dst_v, zbuf)
    _agg_pass(tab, out.at[c], c, s, src_v, dst_v, bufs, zbuf,
              (g0, g1, g2, g3), (s0, s1, s2, s3), acc_sh)


@functools.partial(
    pl.kernel,
    out_type=jax.ShapeDtypeStruct((NC, 2, N_PAD, D), jnp.bfloat16),
    mesh=_mesh,
    scratch_types=_AGG_SCRATCH,
    compiler_params=_sc_params,
)
def _sc_agg2(taba, tabb, src2d, dst2d, out, src_v, dst_v, bufs, zbuf,
             g0, g1, g2, g3, s0, s1, s2, s3, acc_sh):
    c = lax.axis_index("c")
    s = lax.axis_index("s")
    _agg_prologue(c, s, src2d, dst2d, src_v, dst_v, zbuf)
    for half, tab in ((0, taba), (1, tabb)):
        _agg_pass(tab, out.at[c, half], c, s, src_v, dst_v, bufs, zbuf,
                  (g0, g1, g2, g3), (s0, s1, s2, s3), acc_sh)


_BM = 512
_GRID = N_PAD // _BM


def _f32(x):
    return x.astype(jnp.float32)


def _tc_prep_body(degt_ref, x_ref, dis_ref, t1_ref):
    deg = degt_ref[:, 0:1] + degt_ref[:, 1:2]
    dis = lax.rsqrt(deg)
    dis_ref[...] = dis
    t1_ref[...] = (x_ref[...] * dis).astype(jnp.bfloat16)


def _tc_prep(degt, x_pad):
    return pl.pallas_call(
        _tc_prep_body,
        grid=(_GRID,),
        in_specs=[
            pl.BlockSpec((_BM, 2), lambda m: (m, 0)),
            pl.BlockSpec((_BM, D), lambda m: (m, 0)),
        ],
        out_specs=[
            pl.BlockSpec((_BM, 1), lambda m: (m, 0)),
            pl.BlockSpec((_BM, D), lambda m: (m, 0)),
        ],
        out_shape=[
            jax.ShapeDtypeStruct((N_PAD, 1), jnp.float32),
            jax.ShapeDtypeStruct((N_PAD, D), jnp.bfloat16),
        ],
    )(degt, x_pad)


def _tc_layer1_body(p_ref, t1_ref, dis_ref, w1_ref, b1_ref, t2a_ref, t2b_ref):
    dis = dis_ref[...]
    u = (_f32(p_ref[0]) + _f32(p_ref[1]) + _f32(t1_ref[...])) * dis
    h = jnp.dot(u, w1_ref[...], preferred_element_type=jnp.float32) + b1_ref[...]
    h = jnp.maximum(h, 0.0)
    t2a_ref[...] = (h[:, :D] * dis).astype(jnp.bfloat16)
    t2b_ref[...] = (h[:, D:] * dis).astype(jnp.bfloat16)


def _tc_layer1(p, t1, dis, w1, b1):
    hspec = pl.BlockSpec((_BM, D), lambda m: (m, 0))
    hshape = jax.ShapeDtypeStruct((N_PAD, D), jnp.bfloat16)
    return pl.pallas_call(
        _tc_layer1_body,
        grid=(_GRID,),
        in_specs=[
            pl.BlockSpec((NC, _BM, D), lambda m: (0, m, 0)),
            hspec,
            pl.BlockSpec((_BM, 1), lambda m: (m, 0)),
            pl.BlockSpec((D, 2 * D), lambda m: (0, 0)),
            pl.BlockSpec((1, 2 * D), lambda m: (0, 0)),
        ],
        out_specs=[hspec, hspec],
        out_shape=[hshape, hshape],
    )(p, t1, dis, w1, b1)


def _tc_out_body(q_ref, t2a_ref, t2b_ref, dis_ref,
                 wmu_ref, bmu_ref, wls_ref, bls_ref, mu_ref, ls_ref):
    dis = dis_ref[...]
    va = (_f32(q_ref[0, 0]) + _f32(q_ref[1, 0]) + _f32(t2a_ref[...])) * dis
    vb = (_f32(q_ref[0, 1]) + _f32(q_ref[1, 1]) + _f32(t2b_ref[...])) * dis

    def mm(w_ref, b_ref):
        return (jnp.dot(va, w_ref[0], preferred_element_type=jnp.float32)
                + jnp.dot(vb, w_ref[1], preferred_element_type=jnp.float32)
                + b_ref[...])

    mu_ref[...] = mm(wmu_ref, bmu_ref)
    ls_ref[...] = mm(wls_ref, bls_ref)


def _tc_out(q, t2a, t2b, dis, wmu, bmu, wls, bls):
    hspec = pl.BlockSpec((_BM, D), lambda m: (m, 0))
    wspec = pl.BlockSpec((2, D, D), lambda m: (0, 0, 0))
    bspec = pl.BlockSpec((1, D), lambda m: (0, 0))
    return pl.pallas_call(
        _tc_out_body,
        grid=(_GRID,),
        in_specs=[
            pl.BlockSpec((NC, 2, _BM, D), lambda m: (0, 0, m, 0)),
            hspec,
            hspec,
            pl.BlockSpec((_BM, 1), lambda m: (m, 0)),
            wspec, bspec, wspec, bspec,
        ],
        out_specs=[hspec, hspec],
        out_shape=[
            jax.ShapeDtypeStruct((N_PAD, D), jnp.float32),
            jax.ShapeDtypeStruct((N_PAD, D), jnp.float32),
        ],
    )(q, t2a, t2b, dis, wmu, bmu, wls, bls)


def kernel(x, edge_index, W1, b1, Wmu, bmu, Wls, bls):
    src = edge_index[0]
    dst = edge_index[1]
    pad = E_PAD - E
    src_pad = jnp.concatenate([src, jnp.zeros((pad,), jnp.int32)])
    dst_pad = jnp.concatenate([dst, jnp.full((pad,), N, jnp.int32)])
    src2d = src_pad.reshape(E_CHUNKS, 64)
    dst2d = dst_pad.reshape(E_CHUNKS, 64)
    x_pad = jnp.concatenate([x, jnp.zeros((N_PAD - N, D), x.dtype)])

    degp = _sc_deg(dst2d)
    dis, t1 = _tc_prep(degp.T, x_pad)

    p = _sc_agg1(t1, src2d, dst2d)
    t2a, t2b = _tc_layer1(p, t1, dis, W1, b1.reshape(1, 2 * D))

    q = _sc_agg2(t2a, t2b, src2d, dst2d)
    mu, ls = _tc_out(q, t2a, t2b, dis,
                     Wmu.reshape(2, D, D), bmu.reshape(1, D),
                     Wls.reshape(2, D, D), bls.reshape(1, D))
    return (mu[:N], ls[:N])


# speedup vs baseline: 1.7941x; 1.0007x over previous
import functools

import jax
import jax.numpy as jnp
from jax import lax
from jax.experimental import pallas as pl
from jax.experimental.pallas import tpu as pltpu
from jax.experimental.pallas import tpu_sc as plsc

N = 10000
E = 320000
D = 128

NC = 2
NS = 16
NW = NC * NS

N_PAD = 10240
E_PAD = 327680
E_CHUNKS = E_PAD // 64
CH_PER_TILE = E_PAD // NC // NS // 64
NODES_PER_TILE = N_PAD // NS

_mesh = plsc.VectorSubcoreMesh(core_axis_name="c", subcore_axis_name="s",
                               num_cores=NC, num_subcores=NS)
_sc_params = pltpu.CompilerParams(use_tc_tiling_on_sc=False)


@functools.partial(
    pl.kernel,
    out_type=jax.ShapeDtypeStruct((NC, N_PAD), jnp.float32),
    mesh=_mesh,
    scratch_types=[
        pltpu.VMEM((4, 64), jnp.int32),
        pltpu.VMEM((64,), jnp.float32),
        pltpu.VMEM((NODES_PER_TILE,), jnp.float32),
        pltpu.SemaphoreType.DMA,
        pltpu.SemaphoreType.DMA,
        pltpu.SemaphoreType.DMA,
        pltpu.SemaphoreType.DMA,
        pltpu.SemaphoreType.DMA,
        pltpu.SemaphoreType.DMA,
        pltpu.SemaphoreType.DMA,
        pltpu.SemaphoreType.DMA,
        pltpu.VMEM_SHARED((N_PAD,), jnp.float32),
    ],
    compiler_params=_sc_params,
)
def _sc_deg(dst2d, degp_out, idx_v, ones_v, buf_v,
            g0, g1, g2, g3, s0, s1, s2, s3, deg_sh):
    c = lax.axis_index("c")
    s = lax.axis_index("s")
    gs = (g0, g1, g2, g3)
    ss = (s0, s1, s2, s3)

    def fill_ones(i, _):
        ones_v[pl.ds(i * 16, 16)] = jnp.full((16,), 1.0, jnp.float32)
        return 0
    lax.fori_loop(0, 4, fill_ones, 0)

    def fill_half(i, _):
        buf_v[pl.ds(i * 16, 16)] = jnp.full((16,), 0.5, jnp.float32)
        return 0
    lax.fori_loop(0, NODES_PER_TILE // 16, fill_half, 0)
    pltpu.sync_copy(buf_v, deg_sh.at[pl.ds(s * NODES_PER_TILE, NODES_PER_TILE)])
    plsc.subcore_barrier()

    base = c * (NS * CH_PER_TILE) + s * CH_PER_TILE

    def iload(j, k):
        pltpu.async_copy(dst2d.at[base + j], idx_v.at[k], gs[k])

    def iwait(k):
        pltpu.make_async_copy(dst2d.at[base], idx_v.at[k], gs[k]).wait()

    def scat(k):
        pltpu.async_copy(ones_v, deg_sh.at[idx_v.at[k]], ss[k], add=True)

    def swaitd(k):
        pltpu.make_async_copy(ones_v, deg_sh.at[idx_v.at[k]], ss[k]).wait()

    for k in range(4):
        iload(k, k)

    def grp(g, _):
        j0 = 4 * g
        for k in range(4):
            iwait(k)
            scat(k)
        for k in range(4):
            swaitd(k)
            iload(j0 + 4 + k, k)
        return 0
    lax.fori_loop(0, CH_PER_TILE // 4 - 1, grp, 0)

    for k in range(4):
        iwait(k)
        scat(k)
    for k in range(4):
        swaitd(k)
    plsc.subcore_barrier()

    pltpu.sync_copy(deg_sh.at[pl.ds(s * NODES_PER_TILE, NODES_PER_TILE)], buf_v)
    pltpu.sync_copy(buf_v, degp_out.at[c, pl.ds(s * NODES_PER_TILE, NODES_PER_TILE)])


def _agg_pass(tab, out_slot, c, s, src_v, dst_v, bufs, zbuf, gs, ss, acc_sh):
    nbase = s * NODES_PER_TILE

    def zcopy(k, _):
        pltpu.sync_copy(zbuf, acc_sh.at[pl.ds(nbase + k * 128, 128)])
        return 0
    lax.fori_loop(0, NODES_PER_TILE // 128, zcopy, 0)
    plsc.subcore_barrier()

    def gather(j, k):
        pltpu.async_copy(tab.at[src_v.at[j]], bufs.at[k], gs[k])

    def gwait(k):
        pltpu.make_async_copy(tab.at[src_v.at[0]], bufs.at[k], gs[k]).wait()

    def scatter(j, k):
        pltpu.async_copy(bufs.at[k], acc_sh.at[dst_v.at[j]], ss[k], add=True)

    def swait(k):
        pltpu.make_async_copy(bufs.at[k], acc_sh.at[dst_v.at[0]], ss[k]).wait()

    for k in range(4):
        gather(k, k)

    def grp(g, _):
        j0 = 4 * g
        for k in range(4):
            gwait(k)
            scatter(j0 + k, k)
        for k in range(4):
            swait(k)
            gather(j0 + 4 + k, k)
        return 0
    lax.fori_loop(0, CH_PER_TILE // 4 - 1, grp, 0)

    j0 = CH_PER_TILE - 4
    for k in range(4):
        gwait(k)
        scatter(j0 + k, k)
    for k in range(4):
        swait(k)

    plsc.subcore_barrier()

    def wb(k, _):
        pltpu.sync_copy(acc_sh.at[pl.ds(nbase + k * 64, 64)], bufs.at[0])
        pltpu.sync_copy(bufs.at[0], out_slot.at[pl.ds(nbase + k * 64, 64)])
        return 0
    lax.fori_loop(0, NODES_PER_TILE // 64, wb, 0)


_AGG_SCRATCH = [
    pltpu.VMEM((CH_PER_TILE, 64), jnp.int32),
    pltpu.VMEM((CH_PER_TILE, 64), jnp.int32),
    pltpu.VMEM((4, 64, D), jnp.bfloat16),
    pltpu.VMEM((128, D), jnp.bfloat16),
    pltpu.SemaphoreType.DMA,
    pltpu.SemaphoreType.DMA,
    pltpu.SemaphoreType.DMA,
    pltpu.SemaphoreType.DMA,
    pltpu.SemaphoreType.DMA,
    pltpu.SemaphoreType.DMA,
    pltpu.SemaphoreType.DMA,
    pltpu.SemaphoreType.DMA,
    pltpu.VMEM_SHARED((N_PAD, D), jnp.bfloat16),
]


def _agg_prologue(c, s, src2d, dst2d, src_v, dst_v, zbuf):
    def zfill(i, _):
        for jj in range(D // 32):
            zbuf[i, pl.ds(jj * 32, 32)] = jnp.zeros((32,), jnp.bfloat16)
        return 0
    lax.fori_loop(0, 128, zfill, 0)

    ebase = c * (NS * CH_PER_TILE) + s * CH_PER_TILE
    pltpu.sync_copy(src2d.at[pl.ds(ebase, CH_PER_TILE)], src_v)
    pltpu.sync_copy(dst2d.at[pl.ds(ebase, CH_PER_TILE)], dst_v)


@functools.partial(
    pl.kernel,
    out_type=jax.ShapeDtypeStruct((NC, N_PAD, D), jnp.bfloat16),
    mesh=_mesh,
    scratch_types=_AGG_SCRATCH,
    compiler_params=_sc_params,
)
def _sc_agg1(tab, src2d, dst2d, out, src_v, dst_v, bufs, zbuf,
             g0, g1, g2, g3, s0, s1, s2, s3, acc_sh):
    c = lax.axis_index("c")
    s = lax.axis_index("s")
    _agg_prologue(c, s, src2d, dst2d, src_v, dst_v, zbuf)
    _agg_pass(tab, out.at[c], c, s, src_v, dst_v, bufs, zbuf,
              (g0, g1, g2, g3), (s0, s1, s2, s3), acc_sh)


@functools.partial(
    pl.kernel,
    out_type=jax.ShapeDtypeStruct((NC, 2, N_PAD, D), jnp.bfloat16),
    mesh=_mesh,
    scratch_types=_AGG_SCRATCH,
    compiler_params=_sc_params,
)
def _sc_agg2(taba, tabb, src2d, dst2d, out, src_v, dst_v, bufs, zbuf,
             g0, g1, g2, g3, s0, s1, s2, s3, acc_sh):
    c = lax.axis_index("c")
    s = lax.axis_index("s")
    _agg_prologue(c, s, src2d, dst2d, src_v, dst_v, zbuf)
    for half, tab in ((0, taba), (1, tabb)):
        _agg_pass(tab, out.at[c, half], c, s, src_v, dst_v, bufs, zbuf,
                  (g0, g1, g2, g3), (s0, s1, s2, s3), acc_sh)


_BM = 512
_GRID = N_PAD // _BM


def _f32(x):
    return x.astype(jnp.float32)


def _tc_prep_body(degt_ref, x_ref, dis_ref, t1_ref):
    deg = degt_ref[:, 0:1] + degt_ref[:, 1:2]
    dis = lax.rsqrt(deg)
    dis_ref[...] = dis
    t1_ref[...] = (x_ref[...] * dis).astype(jnp.bfloat16)


def _tc_prep(degt, x_pad):
    return pl.pallas_call(
        _tc_prep_body,
        grid=(_GRID,),
        in_specs=[
            pl.BlockSpec((_BM, 2), lambda m: (m, 0)),
            pl.BlockSpec((_BM, D), lambda m: (m, 0)),
        ],
        out_specs=[
            pl.BlockSpec((_BM, 1), lambda m: (m, 0)),
            pl.BlockSpec((_BM, D), lambda m: (m, 0)),
        ],
        out_shape=[
            jax.ShapeDtypeStruct((N_PAD, 1), jnp.float32),
            jax.ShapeDtypeStruct((N_PAD, D), jnp.bfloat16),
        ],
    )(degt, x_pad)


def _tc_layer1_body(p_ref, t1_ref, dis_ref, w1_ref, b1_ref, t2a_ref, t2b_ref):
    dis = dis_ref[...]
    u = (_f32(p_ref[0]) + _f32(p_ref[1]) + _f32(t1_ref[...])) * dis
    h = jnp.dot(u, w1_ref[...], preferred_element_type=jnp.float32) + b1_ref[...]
    h = jnp.maximum(h, 0.0)
    t2a_ref[...] = (h[:, :D] * dis).astype(jnp.bfloat16)
    t2b_ref[...] = (h[:, D:] * dis).astype(jnp.bfloat16)


def _tc_layer1(p, t1, dis, w1, b1):
    hspec = pl.BlockSpec((_BM, D), lambda m: (m, 0))
    hshape = jax.ShapeDtypeStruct((N_PAD, D), jnp.bfloat16)
    return pl.pallas_call(
        _tc_layer1_body,
        grid=(_GRID,),
        in_specs=[
            pl.BlockSpec((NC, _BM, D), lambda m: (0, m, 0)),
            hspec,
            pl.BlockSpec((_BM, 1), lambda m: (m, 0)),
            pl.BlockSpec((D, 2 * D), lambda m: (0, 0)),
            pl.BlockSpec((1, 2 * D), lambda m: (0, 0)),
        ],
        out_specs=[hspec, hspec],
        out_shape=[hshape, hshape],
    )(p, t1, dis, w1, b1)


def _tc_out_body(q_ref, t2a_ref, t2b_ref, dis_ref,
                 wmu_ref, bmu_ref, wls_ref, bls_ref, mu_ref, ls_ref):
    dis = dis_ref[...]
    va = (_f32(q_ref[0, 0]) + _f32(q_ref[1, 0]) + _f32(t2a_ref[...])) * dis
    vb = (_f32(q_ref[0, 1]) + _f32(q_ref[1, 1]) + _f32(t2b_ref[...])) * dis

    def mm(w_ref, b_ref):
        return (jnp.dot(va, w_ref[0], preferred_element_type=jnp.float32)
                + jnp.dot(vb, w_ref[1], preferred_element_type=jnp.float32)
                + b_ref[...])

    mu_ref[...] = mm(wmu_ref, bmu_ref)
    ls_ref[...] = mm(wls_ref, bls_ref)


def _tc_out(q, t2a, t2b, dis, wmu, bmu, wls, bls):
    hspec = pl.BlockSpec((_BM, D), lambda m: (m, 0))
    wspec = pl.BlockSpec((2, D, D), lambda m: (0, 0, 0))
    bspec = pl.BlockSpec((1, D), lambda m: (0, 0))
    return pl.pallas_call(
        _tc_out_body,
        grid=(_GRID,),
        in_specs=[
            pl.BlockSpec((NC, 2, _BM, D), lambda m: (0, 0, m, 0)),
            hspec,
            hspec,
            pl.BlockSpec((_BM, 1), lambda m: (m, 0)),
            wspec, bspec, wspec, bspec,
        ],
        out_specs=[hspec, hspec],
        out_shape=[
            jax.ShapeDtypeStruct((N_PAD, D), jnp.float32),
            jax.ShapeDtypeStruct((N_PAD, D), jnp.float32),
        ],
    )(q, t2a, t2b, dis, wmu, bmu, wls, bls)


def kernel(x, edge_index, W1, b1, Wmu, bmu, Wls, bls):
    src = edge_index[0]
    dst = edge_index[1]
    pad = E_PAD - E
    src_pad = jnp.concatenate([src, jnp.zeros((pad,), jnp.int32)])
    dst_pad = jnp.concatenate([dst, jnp.full((pad,), N, jnp.int32)])
    src2d = src_pad.reshape(E_CHUNKS, 64)
    dst2d = dst_pad.reshape(E_CHUNKS, 64)
    x_pad = jnp.concatenate([x, jnp.zeros((N_PAD - N, D), x.dtype)])

    degp = _sc_deg(dst2d)
    dis, t1 = _tc_prep(degp.T, x_pad)

    p = _sc_agg1(t1, src2d, dst2d)
    t2a, t2b = _tc_layer1(p, t1, dis, W1, b1.reshape(1, 2 * D))

    q = _sc_agg2(t2a, t2b, src2d, dst2d)
    mu, ls = _tc_out(q, t2a, t2b, dis,
                     Wmu.reshape(2, D, D), bmu.reshape(1, D),
                     Wls.reshape(2, D, D), bls.reshape(1, D))
    return (mu[:N], ls[:N])
